# baseline (speedup 1.0000x reference)
import functools

import jax
import jax.numpy as jnp
from jax import lax
from jax.experimental import pallas as pl
from jax.experimental.pallas import tpu as pltpu
from jax.experimental.pallas import tpu_sc as plsc

_VOCAB = 128
_HIDDEN = 128
_BATCH = 4
_SEQ = 8192
_B = _BATCH * _SEQ
_NC = 2
_NS = 16
_NW = _NC * _NS
_BPW = _B // _NW
_CH = 128
_NCHUNK = _BPW // _CH
_NBUF = 4


def _make_emb_kernel():
    mesh = plsc.VectorSubcoreMesh(core_axis_name="c", subcore_axis_name="s")

    @functools.partial(
        pl.kernel,
        mesh=mesh,
        out_type=jax.ShapeDtypeStruct((_B, _HIDDEN), jnp.float32),
        scratch_types=[
            pltpu.VMEM((_NCHUNK, _CH), jnp.int32),
            pltpu.VMEM((_NBUF, _CH, _HIDDEN), jnp.float32),
        ]
        + [pltpu.SemaphoreType.DMA] * (2 * _NBUF),
    )
    def emb(table_hbm, idx_hbm, out_hbm, idx_v, rows_v, *sems):
        gsems = sems[:_NBUF]
        wsems = sems[_NBUF:]
        wid = lax.axis_index("s") * _NC + lax.axis_index("c")
        base = wid * _BPW
        pltpu.sync_copy(idx_hbm.at[pl.ds(wid * _NCHUNK, _NCHUNK)], idx_v)

        def gstart(j):
            return pltpu.async_copy(
                table_hbm.at[idx_v.at[j]],
                rows_v.at[j % _NBUF],
                gsems[j % _NBUF],
            )

        def wstart(j):
            return pltpu.async_copy(
                rows_v.at[j % _NBUF],
                out_hbm.at[pl.ds(base + j * _CH, _CH)],
                wsems[j % _NBUF],
            )

        del wstart
        gcp = {}
        for j in range(_NCHUNK):
            if j - _NBUF >= 0:
                gcp[j - _NBUF].wait()
            gcp[j] = gstart(j)
        for j in range(_NCHUNK - _NBUF, _NCHUNK):
            gcp[j].wait()
        pltpu.sync_copy(
            rows_v.at[0], out_hbm.at[pl.ds(base, _CH)]
        )

    return emb


_emb = _make_emb_kernel()


def kernel(input_ids, embed_weight):
    ids = input_ids.reshape(_B // _CH, _CH).astype(jnp.int32)
    out = _emb(embed_weight, ids)
    hidden = out.reshape(_BATCH, _SEQ, _HIDDEN)
    return (hidden, hidden)


# speedup vs baseline: 1.4988x; 1.4988x over previous
import functools

import jax
import jax.numpy as jnp
from jax import lax
from jax.experimental import pallas as pl
from jax.experimental.pallas import tpu as pltpu
from jax.experimental.pallas import tpu_sc as plsc

_VOCAB = 128
_HIDDEN = 128
_BATCH = 4
_SEQ = 8192
_B = _BATCH * _SEQ
_NC = 2
_NS = 16
_NW = _NC * _NS
_BPW = _B // _NW
_CH = 128
_NCHUNK = _BPW // _CH
_NBUF = 4


def _make_emb_kernel():
    mesh = plsc.VectorSubcoreMesh(core_axis_name="c", subcore_axis_name="s")

    @functools.partial(
        pl.kernel,
        mesh=mesh,
        out_type=jax.ShapeDtypeStruct((_B, _HIDDEN), jnp.float32),
        scratch_types=[
            pltpu.VMEM((_NCHUNK, _CH), jnp.int32),
            pltpu.VMEM((_NBUF, _CH, _HIDDEN), jnp.float32),
            pltpu.VMEM_SHARED((_VOCAB, _HIDDEN), jnp.float32),
        ]
        + [pltpu.SemaphoreType.DMA] * (2 * _NBUF),
    )
    def emb(table_hbm, idx_hbm, out_hbm, idx_v, rows_v, table_sh, *sems):
        gsems = sems[:_NBUF]
        wsems = sems[_NBUF:]
        sid = lax.axis_index("s")
        wid = sid * _NC + lax.axis_index("c")
        base = wid * _BPW

        @pl.when(sid == 0)
        def _():
            pltpu.sync_copy(table_hbm, table_sh)

        pltpu.sync_copy(idx_hbm.at[pl.ds(wid * _NCHUNK, _NCHUNK)], idx_v)
        plsc.subcore_barrier()

        def gstart(j):
            return pltpu.async_copy(
                table_sh.at[idx_v.at[j]],
                rows_v.at[j % _NBUF],
                gsems[j % _NBUF],
            )

        def wstart(j):
            return pltpu.async_copy(
                rows_v.at[j % _NBUF],
                out_hbm.at[pl.ds(base + j * _CH, _CH)],
                wsems[j % _NBUF],
            )

        gcp = {j: gstart(j) for j in range(_NBUF - 1)}
        wcp = {}
        for j in range(_NCHUNK):
            gcp[j].wait()
            wcp[j] = wstart(j)
            nj = j + _NBUF - 1
            if nj < _NCHUNK:
                if nj - _NBUF >= 0:
                    wcp[nj - _NBUF].wait()
                gcp[nj] = gstart(nj)
        for j in range(_NCHUNK - _NBUF, _NCHUNK):
            if j >= 0:
                wcp[j].wait()

    return emb


_emb = _make_emb_kernel()


def kernel(input_ids, embed_weight):
    ids = input_ids.reshape(_B // _CH, _CH).astype(jnp.int32)
    out = _emb(embed_weight, ids)
    hidden = out.reshape(_BATCH, _SEQ, _HIDDEN)
    return (hidden, hidden)


# speedup vs baseline: 1.5005x; 1.0011x over previous
import functools

import jax
import jax.numpy as jnp
from jax import lax
from jax.experimental import pallas as pl
from jax.experimental.pallas import tpu as pltpu
from jax.experimental.pallas import tpu_sc as plsc

_VOCAB = 128
_HIDDEN = 128
_BATCH = 4
_SEQ = 8192
_B = _BATCH * _SEQ
_NC = 2
_NS = 16
_NW = _NC * _NS
_BPW = _B // _NW
_CH = 128
_NCHUNK = _BPW // _CH
_NBUF = 4


def _make_emb_kernel():
    mesh = plsc.VectorSubcoreMesh(core_axis_name="c", subcore_axis_name="s")

    @functools.partial(
        pl.kernel,
        mesh=mesh,
        out_type=jax.ShapeDtypeStruct((_B, _HIDDEN), jnp.float32),
        scratch_types=[
            pltpu.VMEM((_BPW,), jnp.int32),
            pltpu.VMEM((_NBUF, _CH, _HIDDEN), jnp.float32),
            pltpu.VMEM_SHARED((_VOCAB, _HIDDEN), jnp.float32),
        ]
        + [pltpu.SemaphoreType.DMA] * (2 * _NBUF),
    )
    def emb(table_hbm, idx_hbm, out_hbm, idx_v, rows_v, table_sh, *sems):
        gsems = sems[:_NBUF]
        wsems = sems[_NBUF:]
        sid = lax.axis_index("s")
        wid = sid * _NC + lax.axis_index("c")
        base = wid * _BPW

        @pl.when(sid == 0)
        def _():
            pltpu.sync_copy(table_hbm, table_sh)

        pltpu.sync_copy(
            idx_hbm.at[wid // 8, pl.ds((wid % 8) * _BPW, _BPW)], idx_v
        )
        plsc.subcore_barrier()

        def gstart(j):
            return pltpu.async_copy(
                table_sh.at[idx_v.at[pl.ds(j * _CH, _CH)]],
                rows_v.at[j % _NBUF],
                gsems[j % _NBUF],
            )

        def wstart(j):
            return pltpu.async_copy(
                rows_v.at[j % _NBUF],
                out_hbm.at[pl.ds(base + j * _CH, _CH)],
                wsems[j % _NBUF],
            )

        gcp = {j: gstart(j) for j in range(_NBUF - 1)}
        wcp = {}
        for j in range(_NCHUNK):
            gcp[j].wait()
            wcp[j] = wstart(j)
            nj = j + _NBUF - 1
            if nj < _NCHUNK:
                if nj - _NBUF >= 0:
                    wcp[nj - _NBUF].wait()
                gcp[nj] = gstart(nj)
        for j in range(_NCHUNK - _NBUF, _NCHUNK):
            if j >= 0:
                wcp[j].wait()

    return emb


_emb = _make_emb_kernel()


def kernel(input_ids, embed_weight):
    ids = input_ids.astype(jnp.int32)
    out = _emb(embed_weight, ids)
    hidden = out.reshape(_BATCH, _SEQ, _HIDDEN)
    return (hidden, hidden)
